# Initial kernel scaffold; baseline (speedup 1.0000x reference)
import jax
import jax.numpy as jnp
from jax.experimental import pallas as pl


def kernel(x_nchw, conv_w, bn_gamma, bn_beta):
    raise NotImplementedError("write your pallas kernel here")


# speedup vs baseline: 1.0293x; 1.0293x over previous
import functools

import jax
import jax.numpy as jnp
from jax import lax
from jax.experimental import pallas as pl
from jax.experimental.pallas import tpu as pltpu

_BN_EPS = 1e-5
_VMEM_LIMIT = 48 * 1024 * 1024


def _stats_kernel(x_ref, g_ref, s_ref):
    i = pl.program_id(1)

    @pl.when(i == 0)
    def _init():
        g_ref[...] = jnp.zeros_like(g_ref)
        s_ref[...] = jnp.zeros_like(s_ref)

    x = x_ref[0]
    xb = x.astype(jnp.bfloat16)
    g_ref[0] += lax.dot_general(xb, xb, (((1,), (1,)), ((), ())),
                                preferred_element_type=jnp.float32)
    s_ref[0] += jnp.sum(x, axis=1, keepdims=True)


def _apply_kernel(x_ref, w_ref, b_ref, o_ref):
    xb = x_ref[0].astype(jnp.bfloat16)
    y = jnp.dot(w_ref[...], xb, preferred_element_type=jnp.float32)
    o_ref[0] = y + b_ref[...]


@jax.jit
def _linear_block(x_nchw, conv_w, bn_gamma, bn_beta):
    N, Cin, H, W = x_nchw.shape
    Cout = conv_w.shape[0]
    HW = H * W
    M = N * HW
    inv_m = 1.0 / float(M)

    x3 = x_nchw.reshape(N, Cin, HW)
    w2 = conv_w.reshape(Cout, Cin)

    ncore = 2 if N % 2 == 0 else 1
    per = N // ncore

    g_part, s_part = pl.pallas_call(
        _stats_kernel,
        out_shape=(jax.ShapeDtypeStruct((ncore, Cin, Cin), jnp.float32),
                   jax.ShapeDtypeStruct((ncore, Cin, 1), jnp.float32)),
        grid=(ncore, per),
        in_specs=[pl.BlockSpec((1, Cin, HW), lambda c, i: (c * per + i, 0, 0))],
        out_specs=(pl.BlockSpec((1, Cin, Cin), lambda c, i: (c, 0, 0)),
                   pl.BlockSpec((1, Cin, 1), lambda c, i: (c, 0, 0))),
        compiler_params=pltpu.CompilerParams(
            dimension_semantics=("parallel", "arbitrary"),
            vmem_limit_bytes=_VMEM_LIMIT,
        ),
        cost_estimate=pl.CostEstimate(
            flops=int(2 * M * Cin * Cin + M * Cin),
            transcendentals=0,
            bytes_accessed=int(4 * (N * Cin * HW + ncore * Cin * (Cin + 1))),
        ),
    )(x3)

    G = jnp.sum(g_part, axis=0)
    s = jnp.sum(s_part, axis=0)[:, 0]
    mean = (w2 @ s) * inv_m
    ey2 = jnp.sum((w2 @ G) * w2, axis=1) * inv_m
    var = jnp.maximum(ey2 - mean * mean, 0.0)
    inv_std = lax.rsqrt(var + _BN_EPS)
    scale = bn_gamma * inv_std
    shift = (bn_beta - mean * scale).reshape(Cout, 1)
    w_folded = (w2 * scale[:, None]).astype(jnp.bfloat16)

    out3 = pl.pallas_call(
        _apply_kernel,
        out_shape=jax.ShapeDtypeStruct((N, Cout, HW), jnp.float32),
        grid=(N,),
        in_specs=[
            pl.BlockSpec((1, Cin, HW), lambda n: (n, 0, 0)),
            pl.BlockSpec((Cout, Cin), lambda n: (0, 0)),
            pl.BlockSpec((Cout, 1), lambda n: (0, 0)),
        ],
        out_specs=pl.BlockSpec((1, Cout, HW), lambda n: (n, 0, 0)),
        compiler_params=pltpu.CompilerParams(
            dimension_semantics=("parallel",),
            vmem_limit_bytes=_VMEM_LIMIT,
        ),
        cost_estimate=pl.CostEstimate(
            flops=int(2 * M * Cin * Cout + M * Cout),
            transcendentals=0,
            bytes_accessed=int(4 * (N * (Cin + Cout) * HW + Cout * (Cin + 1))),
        ),
    )(x3, w_folded, shift)

    return out3.reshape(N, Cout, H, W)


def kernel(x_nchw, conv_w, bn_gamma, bn_beta):
    return _linear_block(x_nchw, conv_w, bn_gamma, bn_beta)
